# baseline (speedup 1.0000x reference)
import functools

import jax
import jax.numpy as jnp
from jax import lax
from jax.experimental import pallas as pl
from jax.experimental.pallas import tpu as pltpu
from jax.experimental.pallas import tpu_sc as plsc

VOCAB = 1000000
DIM = 64
L = 200
B = 4096
HID = 1024
LABELS = 1000
EPS = 1e-5

NC = 2
NS = 16
LANES = 16
NW = NC * NS
BPW = B // NW
ROW_UNROLL = 8


PDIM = 128


def _sc_pool_sum(idx, table):
    mesh = plsc.VectorSubcoreMesh(core_axis_name="c", subcore_axis_name="s")

    @functools.partial(
        pl.kernel,
        mesh=mesh,
        out_type=jax.ShapeDtypeStruct((B, DIM), jnp.float32),
        scratch_types=[
            pltpu.VMEM((L, BPW), jnp.int32),
            pltpu.VMEM((BPW, PDIM), jnp.float32),
            pltpu.VMEM((BPW, PDIM), jnp.float32),
            pltpu.VMEM((BPW, DIM), jnp.float32),
            pltpu.SemaphoreType.DMA,
            pltpu.SemaphoreType.DMA,
        ],
    )
    def sc_kernel(idx_hbm, table_hbm, out_hbm, idx_v, g0, g1, acc, sem0, sem1):
        wid = lax.axis_index("s") * NC + lax.axis_index("c")
        base = wid * BPW

        pltpu.sync_copy(idx_hbm.at[:, pl.ds(base, BPW)], idx_v)

        def issue(l, gbuf, sem):
            pltpu.async_copy(table_hbm.at[idx_v.at[l]], gbuf, sem)

        def drain(gbuf, sem):
            pltpu.make_async_copy(table_hbm.at[idx_v.at[0]], gbuf, sem).wait()

        def accum(gbuf, first):
            @pl.loop(0, BPW, step=ROW_UNROLL)
            def _(r):
                for rr in range(ROW_UNROLL):
                    for c in range(DIM // LANES):
                        sl = (pl.ds(r + rr, 1), pl.ds(LANES * c, LANES))
                        if first:
                            acc[sl] = gbuf[sl]
                        else:
                            acc[sl] = acc[sl] + gbuf[sl]

        issue(0, g0, sem0)
        issue(1, g1, sem1)

        drain(g0, sem0)
        accum(g0, first=True)
        issue(2, g0, sem0)
        drain(g1, sem1)
        accum(g1, first=False)
        issue(3, g1, sem1)

        @pl.loop(0, (L - 4) // 2)
        def _(k):
            drain(g0, sem0)
            accum(g0, first=False)
            issue(2 * k + 4, g0, sem0)
            drain(g1, sem1)
            accum(g1, first=False)
            issue(2 * k + 5, g1, sem1)

        drain(g0, sem0)
        accum(g0, first=False)
        drain(g1, sem1)
        accum(g1, first=False)

        pltpu.sync_copy(acc, out_hbm.at[pl.ds(base, BPW)])

    return sc_kernel(idx, table)


VCHUNK = 4096


def _relayout_body(tt_ref, out_ref):
    tt = tt_ref[...]
    eye = (lax.broadcasted_iota(jnp.int32, (DIM, DIM), 0)
           == lax.broadcasted_iota(jnp.int32, (DIM, DIM), 1)
           ).astype(jnp.float32)
    bt = lax.dot_general(
        tt, eye, (((0,), (0,)), ((), ())),
        preferred_element_type=jnp.float32,
        precision=lax.Precision.HIGHEST,
    )
    out_ref[:, 0:DIM] = bt


def _relayout_table(tableT):
    return pl.pallas_call(
        _relayout_body,
        grid=(pl.cdiv(VOCAB, VCHUNK),),
        in_specs=[pl.BlockSpec((DIM, VCHUNK), lambda i: (0, i))],
        out_specs=pl.BlockSpec((VCHUNK, PDIM), lambda i: (i, 0)),
        out_shape=jax.ShapeDtypeStruct((VOCAB, PDIM), jnp.float32),
    )(tableT)


def _stats_body(cs_ref, w1_ref, b1_ref, gamma_ref, beta_ref, s2_ref, t2_ref):
    cs = cs_ref[...]
    w1 = w1_ref[...]
    mean_c = jnp.sum(cs, axis=0, keepdims=True) * (1.0 / (B * L))
    m2 = lax.dot_general(
        cs, cs, (((0,), (0,)), ((), ())),
        preferred_element_type=jnp.float32,
        precision=lax.Precision.HIGHEST,
    ) * (1.0 / (B * L * L))
    outer = lax.dot_general(
        mean_c, mean_c, (((0,), (0,)), ((), ())),
        preferred_element_type=jnp.float32,
        precision=lax.Precision.HIGHEST,
    )
    cov = m2 - outer
    t = lax.dot_general(
        cov, w1, (((1,), (0,)), ((), ())),
        preferred_element_type=jnp.float32,
        precision=lax.Precision.HIGHEST,
    )
    var = jnp.sum(w1 * t, axis=0, keepdims=True)
    mean_h = lax.dot_general(
        mean_c, w1, (((1,), (0,)), ((), ())),
        preferred_element_type=jnp.float32,
        precision=lax.Precision.HIGHEST,
    ) + b1_ref[...]
    s = gamma_ref[...] * lax.rsqrt(var + EPS)
    s2_ref[...] = s * (1.0 / L)
    t2_ref[...] = b1_ref[...] * s + beta_ref[...] - mean_h * s


def _main_body(cs_ref, w1_ref, w2_ref, b2_ref, s2_ref, t2_ref, out_ref):
    mm = jnp.dot(cs_ref[...], w1_ref[...], preferred_element_type=jnp.float32)
    hn = jnp.maximum(mm * s2_ref[...] + t2_ref[...], 0.0)
    out_ref[...] = (
        jnp.dot(hn, w2_ref[...], preferred_element_type=jnp.float32)
        + b2_ref[...]
    )


BB = 512
NB = B // BB


def kernel(input, W_emb, W1, b1, gamma, beta, W2, b2):
    table_p = _relayout_table(W_emb.T)
    csum = _sc_pool_sum(input, table_p)

    b1r = b1.reshape(1, HID)
    gr = gamma.reshape(1, HID)
    ber = beta.reshape(1, HID)
    b2r = b2.reshape(1, LABELS)

    s2, t2 = pl.pallas_call(
        _stats_body,
        out_shape=[
            jax.ShapeDtypeStruct((1, HID), jnp.float32),
            jax.ShapeDtypeStruct((1, HID), jnp.float32),
        ],
    )(csum, W1, b1r, gr, ber)

    out = pl.pallas_call(
        _main_body,
        grid=(NB,),
        in_specs=[
            pl.BlockSpec((BB, DIM), lambda i: (i, 0)),
            pl.BlockSpec((DIM, HID), lambda i: (0, 0)),
            pl.BlockSpec((HID, LABELS), lambda i: (0, 0)),
            pl.BlockSpec((1, LABELS), lambda i: (0, 0)),
            pl.BlockSpec((1, HID), lambda i: (0, 0)),
            pl.BlockSpec((1, HID), lambda i: (0, 0)),
        ],
        out_specs=pl.BlockSpec((BB, LABELS), lambda i: (i, 0)),
        out_shape=jax.ShapeDtypeStruct((B, LABELS), jnp.float32),
    )(csum, W1, W2, b2r, s2, t2)

    return out


# speedup vs baseline: 1.3963x; 1.3963x over previous
import functools

import jax
import jax.numpy as jnp
from jax import lax
from jax.experimental import pallas as pl
from jax.experimental.pallas import tpu as pltpu
from jax.experimental.pallas import tpu_sc as plsc

VOCAB = 1000000
DIM = 64
L = 200
B = 4096
HID = 1024
LABELS = 1000
EPS = 1e-5

NC = 2
NS = 16
LANES = 16
NW = NC * NS
BPW = B // NW
ROW_UNROLL = 8


PDIM = 128


def _sc_pool_sum(idx, table):
    mesh = plsc.VectorSubcoreMesh(core_axis_name="c", subcore_axis_name="s")

    @functools.partial(
        pl.kernel,
        mesh=mesh,
        out_type=jax.ShapeDtypeStruct((B, DIM), jnp.float32),
        scratch_types=[
            pltpu.VMEM((L, BPW), jnp.int32),
            pltpu.VMEM((BPW, PDIM), jnp.float32),
            pltpu.VMEM((BPW, PDIM), jnp.float32),
            pltpu.VMEM((BPW, DIM), jnp.float32),
            pltpu.SemaphoreType.DMA,
            pltpu.SemaphoreType.DMA,
        ],
    )
    def sc_kernel(idx_hbm, table_hbm, out_hbm, idx_v, g0, g1, acc, sem0, sem1):
        wid = lax.axis_index("s") * NC + lax.axis_index("c")
        base = wid * BPW

        pltpu.sync_copy(idx_hbm.at[:, pl.ds(base, BPW)], idx_v)

        def issue(l, gbuf, sem):
            pltpu.async_copy(table_hbm.at[idx_v.at[l]], gbuf, sem)

        def drain(gbuf, sem):
            pltpu.make_async_copy(table_hbm.at[idx_v.at[0]], gbuf, sem).wait()

        def accum(gbuf, first):
            @pl.loop(0, BPW, step=ROW_UNROLL)
            def _(r):
                for rr in range(ROW_UNROLL):
                    for c in range(DIM // LANES):
                        sl = (pl.ds(r + rr, 1), pl.ds(LANES * c, LANES))
                        if first:
                            acc[sl] = gbuf[sl]
                        else:
                            acc[sl] = acc[sl] + gbuf[sl]

        issue(0, g0, sem0)
        issue(1, g1, sem1)

        drain(g0, sem0)
        accum(g0, first=True)
        issue(2, g0, sem0)
        drain(g1, sem1)
        accum(g1, first=False)
        issue(3, g1, sem1)

        @pl.loop(0, (L - 4) // 2)
        def _(k):
            drain(g0, sem0)
            accum(g0, first=False)
            issue(2 * k + 4, g0, sem0)
            drain(g1, sem1)
            accum(g1, first=False)
            issue(2 * k + 5, g1, sem1)

        drain(g0, sem0)
        accum(g0, first=False)
        drain(g1, sem1)
        accum(g1, first=False)

        pltpu.sync_copy(acc, out_hbm.at[pl.ds(base, BPW)])

    return sc_kernel(idx, table)


VCHUNK = 8192


def _relayout_body(tt_ref, out_ref):
    tt = tt_ref[...]
    eye = (lax.broadcasted_iota(jnp.int32, (DIM, DIM), 0)
           == lax.broadcasted_iota(jnp.int32, (DIM, DIM), 1)
           ).astype(jnp.float32)
    bt = lax.dot_general(
        tt, eye, (((0,), (0,)), ((), ())),
        preferred_element_type=jnp.float32,
    )
    out_ref[:, 0:DIM] = bt


def _relayout_table(tableT):
    return pl.pallas_call(
        _relayout_body,
        grid=(pl.cdiv(VOCAB, VCHUNK),),
        in_specs=[pl.BlockSpec((DIM, VCHUNK), lambda i: (0, i))],
        out_specs=pl.BlockSpec((VCHUNK, PDIM), lambda i: (i, 0)),
        out_shape=jax.ShapeDtypeStruct((VOCAB, PDIM), jnp.float32),
    )(tableT)


def _stats_body(cs_ref, w1_ref, b1_ref, gamma_ref, beta_ref, s2_ref, t2_ref):
    cs = cs_ref[...]
    w1 = w1_ref[...]
    mean_c = jnp.sum(cs, axis=0, keepdims=True) * (1.0 / (B * L))
    m2 = lax.dot_general(
        cs, cs, (((0,), (0,)), ((), ())),
        preferred_element_type=jnp.float32,
        precision=lax.Precision.HIGHEST,
    ) * (1.0 / (B * L * L))
    outer = lax.dot_general(
        mean_c, mean_c, (((0,), (0,)), ((), ())),
        preferred_element_type=jnp.float32,
        precision=lax.Precision.HIGHEST,
    )
    cov = m2 - outer
    t = lax.dot_general(
        cov, w1, (((1,), (0,)), ((), ())),
        preferred_element_type=jnp.float32,
        precision=lax.Precision.HIGHEST,
    )
    var = jnp.sum(w1 * t, axis=0, keepdims=True)
    mean_h = lax.dot_general(
        mean_c, w1, (((1,), (0,)), ((), ())),
        preferred_element_type=jnp.float32,
        precision=lax.Precision.HIGHEST,
    ) + b1_ref[...]
    s = gamma_ref[...] * lax.rsqrt(var + EPS)
    s2_ref[...] = s * (1.0 / L)
    t2_ref[...] = b1_ref[...] * s + beta_ref[...] - mean_h * s


def _main_body(cs_ref, w1_ref, w2_ref, b2_ref, s2_ref, t2_ref, out_ref):
    mm = jnp.dot(cs_ref[...], w1_ref[...], preferred_element_type=jnp.float32)
    hn = jnp.maximum(mm * s2_ref[...] + t2_ref[...], 0.0)
    out_ref[...] = (
        jnp.dot(hn, w2_ref[...], preferred_element_type=jnp.float32)
        + b2_ref[...]
    )


BB = 512
NB = B // BB


def kernel(input, W_emb, W1, b1, gamma, beta, W2, b2):
    table_p = _relayout_table(W_emb.T)
    csum = _sc_pool_sum(input, table_p)

    b1r = b1.reshape(1, HID)
    gr = gamma.reshape(1, HID)
    ber = beta.reshape(1, HID)
    b2r = b2.reshape(1, LABELS)

    s2, t2 = pl.pallas_call(
        _stats_body,
        out_shape=[
            jax.ShapeDtypeStruct((1, HID), jnp.float32),
            jax.ShapeDtypeStruct((1, HID), jnp.float32),
        ],
    )(csum, W1, b1r, gr, ber)

    out = pl.pallas_call(
        _main_body,
        grid=(NB,),
        in_specs=[
            pl.BlockSpec((BB, DIM), lambda i: (i, 0)),
            pl.BlockSpec((DIM, HID), lambda i: (0, 0)),
            pl.BlockSpec((HID, LABELS), lambda i: (0, 0)),
            pl.BlockSpec((1, LABELS), lambda i: (0, 0)),
            pl.BlockSpec((1, HID), lambda i: (0, 0)),
            pl.BlockSpec((1, HID), lambda i: (0, 0)),
        ],
        out_specs=pl.BlockSpec((BB, LABELS), lambda i: (i, 0)),
        out_shape=jax.ShapeDtypeStruct((B, LABELS), jnp.float32),
    )(csum, W1, W2, b2r, s2, t2)

    return out


# speedup vs baseline: 1.4432x; 1.0335x over previous
import functools

import jax
import jax.numpy as jnp
from jax import lax
from jax.experimental import pallas as pl
from jax.experimental.pallas import tpu as pltpu
from jax.experimental.pallas import tpu_sc as plsc

VOCAB = 1000000
DIM = 64
L = 200
B = 4096
HID = 1024
LABELS = 1000
EPS = 1e-5

NC = 2
NS = 16
LANES = 16
NW = NC * NS
BPW = B // NW
ROW_UNROLL = 8


PDIM = 128


def _sc_pool_sum(idx, table):
    mesh = plsc.VectorSubcoreMesh(core_axis_name="c", subcore_axis_name="s")

    @functools.partial(
        pl.kernel,
        mesh=mesh,
        out_type=jax.ShapeDtypeStruct((B, DIM), jnp.float32),
        scratch_types=[
            pltpu.VMEM((L, BPW), jnp.int32),
            pltpu.VMEM((BPW, PDIM), jnp.float32),
            pltpu.VMEM((BPW, PDIM), jnp.float32),
            pltpu.VMEM((BPW, PDIM), jnp.float32),
            pltpu.VMEM((BPW, PDIM), jnp.float32),
            pltpu.VMEM((BPW, DIM), jnp.float32),
            pltpu.SemaphoreType.DMA,
            pltpu.SemaphoreType.DMA,
            pltpu.SemaphoreType.DMA,
            pltpu.SemaphoreType.DMA,
        ],
    )
    def sc_kernel(idx_hbm, table_hbm, out_hbm, idx_v, ga0, ga1, gb0, gb1,
                  acc, sa0, sa1, sb0, sb1):
        wid = lax.axis_index("s") * NC + lax.axis_index("c")
        base = wid * BPW

        pltpu.sync_copy(idx_hbm.at[:, pl.ds(base, BPW)], idx_v)

        def issue_pair(l, b0, b1, s0, s1):
            pltpu.async_copy(table_hbm.at[idx_v.at[l]], b0, s0)
            pltpu.async_copy(table_hbm.at[idx_v.at[l + 1]], b1, s1)

        def drain_pair(b0, b1, s0, s1):
            pltpu.make_async_copy(table_hbm.at[idx_v.at[0]], b0, s0).wait()
            pltpu.make_async_copy(table_hbm.at[idx_v.at[0]], b1, s1).wait()

        def accum_pair(b0, b1, first):
            @pl.loop(0, BPW, step=ROW_UNROLL)
            def _(r):
                for rr in range(ROW_UNROLL):
                    for c in range(DIM // LANES):
                        sl = (pl.ds(r + rr, 1), pl.ds(LANES * c, LANES))
                        if first:
                            acc[sl] = b0[sl] + b1[sl]
                        else:
                            acc[sl] = acc[sl] + b0[sl] + b1[sl]

        issue_pair(0, ga0, ga1, sa0, sa1)
        issue_pair(2, gb0, gb1, sb0, sb1)

        drain_pair(ga0, ga1, sa0, sa1)
        accum_pair(ga0, ga1, first=True)
        issue_pair(4, ga0, ga1, sa0, sa1)
        drain_pair(gb0, gb1, sb0, sb1)
        accum_pair(gb0, gb1, first=False)
        issue_pair(6, gb0, gb1, sb0, sb1)

        @pl.loop(0, (L - 8) // 4)
        def _(k):
            drain_pair(ga0, ga1, sa0, sa1)
            accum_pair(ga0, ga1, first=False)
            issue_pair(4 * k + 8, ga0, ga1, sa0, sa1)
            drain_pair(gb0, gb1, sb0, sb1)
            accum_pair(gb0, gb1, first=False)
            issue_pair(4 * k + 10, gb0, gb1, sb0, sb1)

        drain_pair(ga0, ga1, sa0, sa1)
        accum_pair(ga0, ga1, first=False)
        drain_pair(gb0, gb1, sb0, sb1)
        accum_pair(gb0, gb1, first=False)

        pltpu.sync_copy(acc, out_hbm.at[pl.ds(base, BPW)])

    return sc_kernel(idx, table)


VCHUNK = 8192


def _relayout_body(tt_ref, out_ref):
    tt = tt_ref[...]
    eye = (lax.broadcasted_iota(jnp.int32, (DIM, DIM), 0)
           == lax.broadcasted_iota(jnp.int32, (DIM, DIM), 1)
           ).astype(jnp.float32)
    bt = lax.dot_general(
        tt, eye, (((0,), (0,)), ((), ())),
        preferred_element_type=jnp.float32,
    )
    out_ref[:, 0:DIM] = bt


def _relayout_table(tableT):
    return pl.pallas_call(
        _relayout_body,
        grid=(pl.cdiv(VOCAB, VCHUNK),),
        in_specs=[pl.BlockSpec((DIM, VCHUNK), lambda i: (0, i))],
        out_specs=pl.BlockSpec((VCHUNK, PDIM), lambda i: (i, 0)),
        out_shape=jax.ShapeDtypeStruct((VOCAB, PDIM), jnp.float32),
    )(tableT)


def _stats_body(cs_ref, w1_ref, b1_ref, gamma_ref, beta_ref, s2_ref, t2_ref):
    cs = cs_ref[...]
    w1 = w1_ref[...]
    mean_c = jnp.sum(cs, axis=0, keepdims=True) * (1.0 / (B * L))
    m2 = lax.dot_general(
        cs, cs, (((0,), (0,)), ((), ())),
        preferred_element_type=jnp.float32,
        precision=lax.Precision.HIGHEST,
    ) * (1.0 / (B * L * L))
    outer = lax.dot_general(
        mean_c, mean_c, (((0,), (0,)), ((), ())),
        preferred_element_type=jnp.float32,
        precision=lax.Precision.HIGHEST,
    )
    cov = m2 - outer
    t = lax.dot_general(
        cov, w1, (((1,), (0,)), ((), ())),
        preferred_element_type=jnp.float32,
        precision=lax.Precision.HIGHEST,
    )
    var = jnp.sum(w1 * t, axis=0, keepdims=True)
    mean_h = lax.dot_general(
        mean_c, w1, (((1,), (0,)), ((), ())),
        preferred_element_type=jnp.float32,
        precision=lax.Precision.HIGHEST,
    ) + b1_ref[...]
    s = gamma_ref[...] * lax.rsqrt(var + EPS)
    s2_ref[...] = s * (1.0 / L)
    t2_ref[...] = b1_ref[...] * s + beta_ref[...] - mean_h * s


def _main_body(cs_ref, w1_ref, w2_ref, b2_ref, s2_ref, t2_ref, out_ref):
    mm = jnp.dot(cs_ref[...], w1_ref[...], preferred_element_type=jnp.float32)
    hn = jnp.maximum(mm * s2_ref[...] + t2_ref[...], 0.0)
    out_ref[...] = (
        jnp.dot(hn, w2_ref[...], preferred_element_type=jnp.float32)
        + b2_ref[...]
    )


BB = 512
NB = B // BB


def kernel(input, W_emb, W1, b1, gamma, beta, W2, b2):
    table_p = _relayout_table(W_emb.T)
    csum = _sc_pool_sum(input, table_p)

    b1r = b1.reshape(1, HID)
    gr = gamma.reshape(1, HID)
    ber = beta.reshape(1, HID)
    b2r = b2.reshape(1, LABELS)

    s2, t2 = pl.pallas_call(
        _stats_body,
        out_shape=[
            jax.ShapeDtypeStruct((1, HID), jnp.float32),
            jax.ShapeDtypeStruct((1, HID), jnp.float32),
        ],
    )(csum, W1, b1r, gr, ber)

    out = pl.pallas_call(
        _main_body,
        grid=(NB,),
        in_specs=[
            pl.BlockSpec((BB, DIM), lambda i: (i, 0)),
            pl.BlockSpec((DIM, HID), lambda i: (0, 0)),
            pl.BlockSpec((HID, LABELS), lambda i: (0, 0)),
            pl.BlockSpec((1, LABELS), lambda i: (0, 0)),
            pl.BlockSpec((1, HID), lambda i: (0, 0)),
            pl.BlockSpec((1, HID), lambda i: (0, 0)),
        ],
        out_specs=pl.BlockSpec((BB, LABELS), lambda i: (i, 0)),
        out_shape=jax.ShapeDtypeStruct((B, LABELS), jnp.float32),
    )(csum, W1, W2, b2r, s2, t2)

    return out
